# baseline (speedup 1.0000x reference)
import jax
import jax.numpy as jnp
from jax.experimental import pallas as pl
from jax.experimental.pallas import tpu as pltpu

BLOCK_C = 2048


def _linear_t_kernel(wbt_ref, xt_ref, o_ref):
    o_ref[...] = jax.lax.dot_general(
        wbt_ref[...], xt_ref[...],
        dimension_numbers=(((0,), (0,)), ((), ())),
        preferred_element_type=jnp.float32,
    )


@jax.jit
def kernel(x, W, b):
    batch, k = x.shape
    num_classes = W.shape[0]
    wbt = jnp.concatenate([W.T, b.reshape(1, num_classes)], axis=0)
    xt1 = jnp.concatenate(
        [x, jnp.ones((batch, 1), jnp.float32)], axis=1
    ).T
    grid = (pl.cdiv(num_classes, BLOCK_C),)
    out_t = pl.pallas_call(
        _linear_t_kernel,
        grid=grid,
        in_specs=[
            pl.BlockSpec((k + 1, BLOCK_C), lambda i: (0, i)),
            pl.BlockSpec((k + 1, batch), lambda i: (0, 0)),
        ],
        out_specs=pl.BlockSpec((BLOCK_C, batch), lambda i: (i, 0)),
        out_shape=jax.ShapeDtypeStruct((num_classes, batch), jnp.float32),
        compiler_params=pltpu.CompilerParams(
            dimension_semantics=("parallel",),
            allow_input_fusion=(True, True),
        ),
    )(wbt, xt1)
    return out_t.T
